# baseline (speedup 1.0000x reference)
import functools

import jax
import jax.numpy as jnp
from jax.experimental import pallas as pl
from jax.experimental.pallas import tpu as pltpu

_NMS_PRE = 1024
_NMS_POST = 256
_NMS_THRESH = 0.1
_B = 4
_K = 1280
_KB = 128
_NBINS = 1024


def _nms_body(a_ref, at_ref, ir_ref, ic_ref, o_ref, prec_ref, sup_ref):
    a = a_ref[0]
    at = at_ref[0]
    i_row = ir_ref[0]
    i_col = ic_ref[0]

    s_row = at[7:8, :]
    v_row = jnp.where(s_row >= 0.0, 1.0, 0.0)

    xc = at[0:1, :]
    yc = at[1:2, :]
    dxc = jnp.abs(at[3:4, :])
    dyc = jnp.abs(at[4:5, :])
    x1c = xc - dxc * 0.5
    x2c = xc + dxc * 0.5
    y1c = yc - dyc * 0.5
    y2c = yc + dyc * 0.5
    areac = dxc * dyc

    for r in range(_K // _KB):
        base = r * _KB
        s_cb = a[base:base + _KB, 7:8]
        i_cb = ic_ref[0][base:base + _KB, :]
        v_cb = jnp.where(s_cb >= 0.0, 1.0, 0.0)
        pr = jnp.where(
            (s_cb > s_row) | ((s_cb == s_row) & (i_cb < i_row)), 1.0, 0.0)
        prec_ref[base:base + _KB, :] = (pr * v_cb * v_row).astype(jnp.bfloat16)

    ones_row = jnp.where(i_row >= -1.0, 1.0, 0.0).astype(jnp.bfloat16)
    rank_row = jnp.dot(ones_row, prec_ref[:, :],
                       preferred_element_type=jnp.float32)
    top_row = jnp.where(rank_row < float(_NMS_PRE), 1.0, 0.0) * v_row

    ri_all = jax.lax.broadcasted_iota(jnp.int32, (_KB, _K), 0)
    ci_all = jax.lax.broadcasted_iota(jnp.int32, (_KB, _K), 1)
    for r in range(_K // _KB):
        base = r * _KB
        xr = a[base:base + _KB, 0:1]
        yr = a[base:base + _KB, 1:2]
        dxr = jnp.abs(a[base:base + _KB, 3:4])
        dyr = jnp.abs(a[base:base + _KB, 4:5])
        x1r = xr - dxr * 0.5
        x2r = xr + dxr * 0.5
        y1r = yr - dyr * 0.5
        y2r = yr + dyr * 0.5
        arear = dxr * dyr
        iw = jnp.clip(jnp.minimum(x2r, x2c) - jnp.maximum(x1r, x1c), 0.0)
        ih = jnp.clip(jnp.minimum(y2r, y2c) - jnp.maximum(y1r, y1c), 0.0)
        inter = iw * ih
        union = arear + areac - inter
        iou = inter / jnp.clip(union, 1e-6)

        precB = prec_ref[base:base + _KB, :].astype(jnp.float32)
        s_cb = a[base:base + _KB, 7:8]
        v_cb = jnp.where(s_cb >= 0.0, 1.0, 0.0)
        vv = v_cb * v_row
        eye = jnp.where(ri_all + base == ci_all, 1.0, 0.0)
        prec2B = vv - precB - eye * vv
        rank_cb = jnp.sum(prec2B, axis=1, keepdims=True)
        top_cb = jnp.where(rank_cb < float(_NMS_PRE), 1.0, 0.0) * v_cb
        supB = jnp.where(iou > _NMS_THRESH, 1.0, 0.0) * precB * top_cb * top_row
        sup_ref[base:base + _KB, :] = supB.astype(jnp.bfloat16)

    it8 = (jax.lax.broadcasted_iota(jnp.int32, (8, _K), 1)
           + jax.lax.broadcasted_iota(jnp.int32, (8, _K), 0))
    ones8 = jnp.where(it8 >= 0, 1.0, 0.0)

    def w_cond(c):
        k_prev, k, it = c
        return (it < _K + 2) & (jnp.sum(jnp.abs(k - k_prev)) > 0.0)

    def w_body(c):
        _, k, it = c
        supp = jnp.dot(k.astype(jnp.bfloat16), sup_ref[:, :],
                       preferred_element_type=jnp.float32)
        k_new = jnp.where(supp < 0.5, 1.0, 0.0)
        return k, k_new, it + 1

    _, keep8, _ = jax.lax.while_loop(
        w_cond, w_body, (-ones8, ones8, jnp.int32(0)))
    keep_f = keep8[0:1, :] * top_row

    pos = jnp.dot(keep_f.astype(jnp.bfloat16), prec_ref[:, :],
                  preferred_element_type=jnp.float32)
    out_r = jax.lax.broadcasted_iota(jnp.int32, (_NMS_POST, _K), 0)
    posi = pos.astype(jnp.int32)
    sel = jnp.where(posi == out_r, 1.0, 0.0) * keep_f

    score_clean = jnp.where(a[:, 7:8] >= 0.0, a[:, 7:8], 0.0)
    payload = jnp.concatenate([a[:, 0:7], score_clean], axis=1)
    o_ref[0] = jnp.dot(sel, payload, preferred_element_type=jnp.float32)


@functools.partial(jax.jit, static_argnames=("interpret",))
def _nms_pallas(a, at, ir, ic, interpret=False):
    return pl.pallas_call(
        _nms_body,
        grid=(_B,),
        in_specs=[
            pl.BlockSpec((1, _K, 8), lambda b: (b, 0, 0)),
            pl.BlockSpec((1, 8, _K), lambda b: (b, 0, 0)),
            pl.BlockSpec((1, 1, _K), lambda b: (b, 0, 0)),
            pl.BlockSpec((1, _K, 1), lambda b: (b, 0, 0)),
        ],
        out_specs=pl.BlockSpec((1, _NMS_POST, 8), lambda b: (b, 0, 0)),
        out_shape=jax.ShapeDtypeStruct((_B, _NMS_POST, 8), jnp.float32),
        scratch_shapes=[
            pltpu.VMEM((_K, _K), jnp.bfloat16),
            pltpu.VMEM((_K, _K), jnp.bfloat16),
        ],
        interpret=interpret,
    )(a, at, ir, ic)


def kernel(batch_box_preds, batch_cls_scores, batch_index, batch_size, interpret=False):
    n = batch_cls_scores.shape[0]
    s = batch_cls_scores
    bi = batch_index.astype(jnp.int32)

    masks = bi[None, :] == jnp.arange(_B, dtype=jnp.int32)[:, None]
    ms = jnp.where(masks, s[None, :], -1.0)
    k32 = jnp.arange(32, dtype=jnp.int32)
    e1 = k32.astype(jnp.float32) * (1.0 / 32.0)
    cge1 = jnp.sum(ms[:, :, None] >= e1[None, None, :], axis=1)
    k1 = jnp.max(jnp.where(cge1 >= _NMS_PRE, k32[None, :], 0), axis=1)
    t1 = k1.astype(jnp.float32) * (1.0 / 32.0)
    e2 = t1[:, None] + k32.astype(jnp.float32)[None, :] * (1.0 / _NBINS)
    cge2 = jnp.sum(ms[:, :, None] >= e2[:, None, :], axis=1)
    j2 = jnp.max(jnp.where(cge2 >= _NMS_PRE, k32[None, :], 0), axis=1)
    t = t1 + j2.astype(jnp.float32) * (1.0 / _NBINS)
    t = jnp.where(jnp.arange(_B) < batch_size, t, 2.0)
    ncand = jnp.take_along_axis(cge2, j2[:, None], axis=1)[:, 0].astype(jnp.int32)
    ncand = jnp.where(jnp.arange(_B) < batch_size, ncand, 0)

    cand = s >= t[bi]
    csum = jnp.cumsum(cand.astype(jnp.int32))
    cstarts = jnp.concatenate(
        [jnp.zeros((1,), jnp.int32), jnp.cumsum(ncand.astype(jnp.int32))[:-1]])
    pos_in_b = csum - 1 - cstarts[bi]
    gpos = jnp.where(cand & (pos_in_b < _K), bi * _K + pos_in_b, _B * _K)
    cs = jnp.full((_B * _K + 1,), -1.0, jnp.float32).at[gpos].set(s)[:-1]
    ci = jnp.zeros((_B * _K + 1,), jnp.int32).at[gpos].set(
        jnp.arange(n, dtype=jnp.int32))[:-1]

    boxes_sel = jnp.take(batch_box_preds, ci, axis=0).reshape(_B, _K, 7)
    cs = cs.reshape(_B, _K)
    cif = ci.reshape(_B, _K).astype(jnp.float32)
    a = jnp.concatenate([boxes_sel, cs[..., None]], axis=-1)
    at = jnp.swapaxes(a, 1, 2)
    ir = cif[:, None, :]
    ic = cif[:, :, None]
    return _nms_pallas(a, at, ir, ic, interpret=interpret)


# speedup vs baseline: 2.6705x; 2.6705x over previous
import jax
import jax.numpy as jnp
from jax.experimental import pallas as pl
from jax.experimental.pallas import tpu as pltpu

_NMS_PRE = 1024
_NMS_POST = 256
_NMS_THRESH = 0.1
_B = 4
_BLK = 128
_NBLK = _NMS_PRE // _BLK


def _nms_body(a_ref, at_ref, o_ref, sup_ref):
    a = a_ref[0]
    at = at_ref[0]

    xc = at[0:1, :]
    yc = at[1:2, :]
    dxc = jnp.abs(at[3:4, :])
    dyc = jnp.abs(at[4:5, :])
    x1c = xc - dxc * 0.5
    x2c = xc + dxc * 0.5
    y1c = yc - dyc * 0.5
    y2c = yc + dyc * 0.5
    areac = dxc * dyc
    scorec = at[7:8, :]

    col_id = jax.lax.broadcasted_iota(jnp.int32, (_BLK, _NMS_PRE), 1)

    for r in range(_NBLK):
        base = r * _BLK
        xr = a[base:base + _BLK, 0:1]
        yr = a[base:base + _BLK, 1:2]
        dxr = jnp.abs(a[base:base + _BLK, 3:4])
        dyr = jnp.abs(a[base:base + _BLK, 4:5])
        x1r = xr - dxr * 0.5
        x2r = xr + dxr * 0.5
        y1r = yr - dyr * 0.5
        y2r = yr + dyr * 0.5
        arear = dxr * dyr

        iw = jnp.clip(jnp.minimum(x2r, x2c) - jnp.maximum(x1r, x1c), 0.0)
        ih = jnp.clip(jnp.minimum(y2r, y2c) - jnp.maximum(y1r, y1c), 0.0)
        inter = iw * ih
        union = arear + areac - inter
        iou = inter / jnp.clip(union, 1e-6)
        row_id = jax.lax.broadcasted_iota(jnp.int32, (_BLK, _NMS_PRE), 0) + base
        sup = jnp.where((iou > _NMS_THRESH) & (col_id > row_id), 1.0, 0.0)
        sup_ref[base:base + _BLK, :] = sup

    it8 = (jax.lax.broadcasted_iota(jnp.int32, (8, _NMS_PRE), 1)
           + jax.lax.broadcasted_iota(jnp.int32, (8, _NMS_PRE), 0))
    ones = jnp.where(it8 >= 0, 1.0, 0.0)

    def w_cond(c):
        k_prev, k, it = c
        return (it < _NMS_PRE + 2) & (jnp.sum(jnp.abs(k - k_prev)) > 0.0)

    def w_body(c):
        _, k, it = c
        supp = jnp.dot(k, sup_ref[:, :], preferred_element_type=jnp.float32)
        k_new = jnp.where(supp < 0.5, 1.0, 0.0)
        return k, k_new, it + 1

    _, keep8, _ = jax.lax.while_loop(
        w_cond, w_body, (-ones, ones, jnp.int32(0)))
    keep = keep8[0:1, :]

    valid = jnp.where(scorec != -jnp.inf, 1.0, 0.0)
    keep_f = keep * valid

    ri = jax.lax.broadcasted_iota(jnp.int32, (_NMS_PRE, _NMS_PRE), 0)
    ci = jax.lax.broadcasted_iota(jnp.int32, (_NMS_PRE, _NMS_PRE), 1)
    tri = jnp.where(ri <= ci, 1.0, 0.0)
    pos = jnp.dot(keep_f, tri, preferred_element_type=jnp.float32)

    out_r = jax.lax.broadcasted_iota(jnp.int32, (_NMS_POST, _NMS_PRE), 0)
    posi = (pos - 1.0).astype(jnp.int32)
    sel = jnp.where(posi == out_r, 1.0, 0.0) * keep_f

    score_clean = jnp.where(a[:, 7:8] != -jnp.inf, a[:, 7:8], 0.0)
    a_mm = jnp.concatenate([a[:, 0:7], score_clean], axis=1)
    o_ref[0] = jnp.dot(sel, a_mm, preferred_element_type=jnp.float32)


@jax.jit
def _nms_pallas(a, at):
    return pl.pallas_call(
        _nms_body,
        grid=(_B,),
        in_specs=[
            pl.BlockSpec((1, _NMS_PRE, 8), lambda b: (b, 0, 0)),
            pl.BlockSpec((1, 8, _NMS_PRE), lambda b: (b, 0, 0)),
        ],
        out_specs=pl.BlockSpec((1, _NMS_POST, 8), lambda b: (b, 0, 0)),
        out_shape=jax.ShapeDtypeStruct((_B, _NMS_POST, 8), jnp.float32),
        scratch_shapes=[
            pltpu.VMEM((_NMS_PRE, _NMS_PRE), jnp.float32),
        ],
    )(a, at)


def kernel(batch_box_preds, batch_cls_scores, batch_index, batch_size):
    n = batch_cls_scores.shape[0]
    pad_b = jnp.full((_NMS_PRE,), 127, batch_index.dtype)
    pad_s = jnp.full((_NMS_PRE,), jnp.inf, jnp.float32)
    pad_i = jnp.zeros((_NMS_PRE,), jnp.int32)
    bi_p = jnp.concatenate([batch_index, pad_b])
    ns_p = jnp.concatenate([-batch_cls_scores, pad_s])
    ix_p = jnp.concatenate([jnp.arange(n, dtype=jnp.int32), pad_i])
    _, s_neg, s_idx = jax.lax.sort((bi_p, ns_p, ix_p), num_keys=2, is_stable=True)

    bids = jnp.arange(_B + 1, dtype=batch_index.dtype)
    starts = jnp.searchsorted(batch_index, bids).astype(jnp.int32)
    lane = jnp.arange(_NMS_PRE, dtype=jnp.int32)

    tops, idxs = [], []
    for b in range(_B):
        sc = -jax.lax.dynamic_slice(s_neg, (starts[b],), (_NMS_PRE,))
        ix = jax.lax.dynamic_slice(s_idx, (starts[b],), (_NMS_PRE,))
        m = (lane < (starts[b + 1] - starts[b])) & (b < batch_size)
        tops.append(jnp.where(m, sc, -jnp.inf))
        idxs.append(ix)
    top_scores = jnp.stack(tops)
    top_idx = jnp.stack(idxs)
    boxes_sel = jnp.take(batch_box_preds, top_idx.reshape(-1), axis=0)
    boxes_sel = boxes_sel.reshape(_B, _NMS_PRE, 7)
    a = jnp.concatenate([boxes_sel, top_scores[..., None]], axis=-1)
    at = jnp.swapaxes(a, 1, 2)
    return _nms_pallas(a, at)
